# Initial kernel scaffold; baseline (speedup 1.0000x reference)
import jax
import jax.numpy as jnp
from jax.experimental import pallas as pl


def kernel(x):
    raise NotImplementedError("write your pallas kernel here")


# speedup vs baseline: 1.3048x; 1.3048x over previous
import functools

import numpy as np
import jax
import jax.numpy as jnp
from jax.experimental import pallas as pl

_ROTATIONS = ((13, 15, 26, 6), (17, 29, 16, 24))
_PARITY = np.uint32(0x1BD11BDA)
_THRESH = 1677721


def _np_threefry2x32(k0, k1, x0, x1):
    k0 = np.uint32(k0)
    k1 = np.uint32(k1)
    ks = (k0, k1, np.uint32(k0 ^ k1 ^ _PARITY))
    x0 = (x0 + ks[0]).astype(np.uint32)
    x1 = (x1 + ks[1]).astype(np.uint32)
    for i in range(5):
        for r in _ROTATIONS[i % 2]:
            x0 = (x0 + x1).astype(np.uint32)
            x1 = ((x1 << np.uint32(r)) | (x1 >> np.uint32(32 - r))).astype(np.uint32)
            x1 = x1 ^ x0
        x0 = (x0 + ks[(i + 1) % 3]).astype(np.uint32)
        x1 = (x1 + ks[(i + 2) % 3] + np.uint32(i + 1)).astype(np.uint32)
    return x0, x1


def _child_keys(seed, num):
    lo = np.arange(num, dtype=np.uint32)
    hi = np.zeros(num, dtype=np.uint32)
    y0, y1 = _np_threefry2x32(np.uint32(seed >> 32), np.uint32(seed & 0xFFFFFFFF), hi, lo)
    return np.stack([y0, y1], axis=-1)


_KEYS = _child_keys(42, 4)


def _mask_mul_kernel(x_ref, o_ref, *, bs, C):
    b = pl.program_id(0)
    i = pl.program_id(1)

    k0 = jnp.uint32(_KEYS[0, 0])
    k1 = jnp.uint32(_KEYS[0, 1])
    for bb in range(1, _KEYS.shape[0]):
        k0 = jnp.where(b == bb, jnp.uint32(_KEYS[bb, 0]), k0)
        k1 = jnp.where(b == bb, jnp.uint32(_KEYS[bb, 1]), k1)
    ks2 = k0 ^ k1 ^ _PARITY
    ks = (k0, k1, ks2)

    row = jax.lax.broadcasted_iota(jnp.int32, (bs, C), 0)
    col = jax.lax.broadcasted_iota(jnp.int32, (bs, C), 1)
    lo = ((row + i * bs) * C + col).astype(jnp.uint32)

    x1 = lo + ks[1]
    x0 = jnp.full((bs, C), jnp.uint32(0), dtype=jnp.uint32) + ks[0]
    for r in range(5):
        for rot in _ROTATIONS[r % 2]:
            x0 = x0 + x1
            x1 = ((x1 << jnp.uint32(rot)) | (x1 >> jnp.uint32(32 - rot))) ^ x0
        x0 = x0 + ks[(r + 1) % 3]
        x1 = x1 + (ks[(r + 2) % 3] + jnp.uint32(r + 1))

    bits = x0 ^ x1
    keep = (bits >> jnp.uint32(9)).astype(jnp.int32) > jnp.int32(_THRESH)
    o_ref[0] = jnp.where(keep, x_ref[0], jnp.float32(0.0))


@jax.jit
def kernel(x):
    B, S, C = x.shape
    bs = 256
    grid = (B, S // bs)
    return pl.pallas_call(
        functools.partial(_mask_mul_kernel, bs=bs, C=C),
        grid=grid,
        in_specs=[pl.BlockSpec((1, bs, C), lambda b, i: (b, i, 0))],
        out_specs=pl.BlockSpec((1, bs, C), lambda b, i: (b, i, 0)),
        out_shape=jax.ShapeDtypeStruct((B, S, C), x.dtype),
    )(x)


# speedup vs baseline: 1.3062x; 1.0011x over previous
import functools

import numpy as np
import jax
import jax.numpy as jnp
from jax.experimental import pallas as pl

_ROTATIONS = ((13, 15, 26, 6), (17, 29, 16, 24))
_PARITY = np.uint32(0x1BD11BDA)
_THRESH = 1677721


def _np_threefry2x32(k0, k1, x0, x1):
    k0 = np.uint32(k0)
    k1 = np.uint32(k1)
    ks = (k0, k1, np.uint32(k0 ^ k1 ^ _PARITY))
    x0 = (x0 + ks[0]).astype(np.uint32)
    x1 = (x1 + ks[1]).astype(np.uint32)
    for i in range(5):
        for r in _ROTATIONS[i % 2]:
            x0 = (x0 + x1).astype(np.uint32)
            x1 = ((x1 << np.uint32(r)) | (x1 >> np.uint32(32 - r))).astype(np.uint32)
            x1 = x1 ^ x0
        x0 = (x0 + ks[(i + 1) % 3]).astype(np.uint32)
        x1 = (x1 + ks[(i + 2) % 3] + np.uint32(i + 1)).astype(np.uint32)
    return x0, x1


def _child_keys(seed, num):
    lo = np.arange(num, dtype=np.uint32)
    hi = np.zeros(num, dtype=np.uint32)
    y0, y1 = _np_threefry2x32(np.uint32(seed >> 32), np.uint32(seed & 0xFFFFFFFF), hi, lo)
    return np.stack([y0, y1], axis=-1)


_KEYS = _child_keys(42, 4)


def _mask_mul_kernel(x_ref, o_ref, *, bs, C):
    b = pl.program_id(0)
    i = pl.program_id(1)

    k0 = jnp.uint32(_KEYS[0, 0])
    k1 = jnp.uint32(_KEYS[0, 1])
    for bb in range(1, _KEYS.shape[0]):
        k0 = jnp.where(b == bb, jnp.uint32(_KEYS[bb, 0]), k0)
        k1 = jnp.where(b == bb, jnp.uint32(_KEYS[bb, 1]), k1)
    ks2 = k0 ^ k1 ^ _PARITY
    ks = (k0, k1, ks2)

    row = jax.lax.broadcasted_iota(jnp.int32, (bs, C), 0)
    col = jax.lax.broadcasted_iota(jnp.int32, (bs, C), 1)
    lo = ((row + i * bs) * C + col).astype(jnp.uint32)

    x1 = lo + ks[1]
    x0 = jnp.full((bs, C), jnp.uint32(0), dtype=jnp.uint32) + ks[0]
    for r in range(5):
        for rot in _ROTATIONS[r % 2]:
            x0 = x0 + x1
            x1 = ((x1 << jnp.uint32(rot)) | (x1 >> jnp.uint32(32 - rot))) ^ x0
        x0 = x0 + ks[(r + 1) % 3]
        x1 = x1 + (ks[(r + 2) % 3] + jnp.uint32(r + 1))

    bits = x0 ^ x1
    keep = (bits >> jnp.uint32(9)).astype(jnp.int32) > jnp.int32(_THRESH)
    o_ref[0] = jnp.where(keep, x_ref[0], jnp.float32(0.0))


@jax.jit
def kernel(x):
    B, S, C = x.shape
    bs = 512
    grid = (B, S // bs)
    return pl.pallas_call(
        functools.partial(_mask_mul_kernel, bs=bs, C=C),
        grid=grid,
        in_specs=[pl.BlockSpec((1, bs, C), lambda b, i: (b, i, 0))],
        out_specs=pl.BlockSpec((1, bs, C), lambda b, i: (b, i, 0)),
        out_shape=jax.ShapeDtypeStruct((B, S, C), x.dtype),
    )(x)


# speedup vs baseline: 1.3115x; 1.0040x over previous
import functools

import numpy as np
import jax
import jax.numpy as jnp
from jax.experimental import pallas as pl

_ROTATIONS = ((13, 15, 26, 6), (17, 29, 16, 24))
_PARITY = np.uint32(0x1BD11BDA)
_THRESH = 1677721


def _np_threefry2x32(k0, k1, x0, x1):
    k0 = np.uint32(k0)
    k1 = np.uint32(k1)
    ks = (k0, k1, np.uint32(k0 ^ k1 ^ _PARITY))
    x0 = (x0 + ks[0]).astype(np.uint32)
    x1 = (x1 + ks[1]).astype(np.uint32)
    for i in range(5):
        for r in _ROTATIONS[i % 2]:
            x0 = (x0 + x1).astype(np.uint32)
            x1 = ((x1 << np.uint32(r)) | (x1 >> np.uint32(32 - r))).astype(np.uint32)
            x1 = x1 ^ x0
        x0 = (x0 + ks[(i + 1) % 3]).astype(np.uint32)
        x1 = (x1 + ks[(i + 2) % 3] + np.uint32(i + 1)).astype(np.uint32)
    return x0, x1


def _child_keys(seed, num):
    lo = np.arange(num, dtype=np.uint32)
    hi = np.zeros(num, dtype=np.uint32)
    y0, y1 = _np_threefry2x32(np.uint32(seed >> 32), np.uint32(seed & 0xFFFFFFFF), hi, lo)
    return np.stack([y0, y1], axis=-1)


_KEYS = _child_keys(42, 4)


def _mask_mul_kernel(pat_ref, x_ref, o_ref, *, bs, C):
    b = pl.program_id(0)
    i = pl.program_id(1)

    k0 = jnp.uint32(_KEYS[0, 0])
    k1 = jnp.uint32(_KEYS[0, 1])
    for bb in range(1, _KEYS.shape[0]):
        k0 = jnp.where(b == bb, jnp.uint32(_KEYS[bb, 0]), k0)
        k1 = jnp.where(b == bb, jnp.uint32(_KEYS[bb, 1]), k1)
    ks2 = k0 ^ k1 ^ _PARITY
    ks = (k0, k1, ks2)

    base_plus_k1 = ks[1] + jnp.uint32(i * (bs * C))

    x1 = pat_ref[0] + base_plus_k1
    x0 = jnp.full((bs, C), jnp.uint32(0), dtype=jnp.uint32) + ks[0]
    for r in range(5):
        for rot in _ROTATIONS[r % 2]:
            x0 = x0 + x1
            x1 = ((x1 << jnp.uint32(rot)) | (x1 >> jnp.uint32(32 - rot))) ^ x0
        x0 = x0 + ks[(r + 1) % 3]
        x1 = x1 + (ks[(r + 2) % 3] + jnp.uint32(r + 1))

    bits = x0 ^ x1
    keep = (bits >> jnp.uint32(9)).astype(jnp.int32) > jnp.int32(_THRESH)
    o_ref[0] = jnp.where(keep, x_ref[0], jnp.float32(0.0))


@jax.jit
def kernel(x):
    B, S, C = x.shape
    bs = 512
    grid = (B, S // bs)
    pattern = (
        jax.lax.broadcasted_iota(jnp.uint32, (1, bs, C), 1) * jnp.uint32(C)
        + jax.lax.broadcasted_iota(jnp.uint32, (1, bs, C), 2)
    )
    return pl.pallas_call(
        functools.partial(_mask_mul_kernel, bs=bs, C=C),
        grid=grid,
        in_specs=[
            pl.BlockSpec((1, bs, C), lambda b, i: (0, 0, 0)),
            pl.BlockSpec((1, bs, C), lambda b, i: (b, i, 0)),
        ],
        out_specs=pl.BlockSpec((1, bs, C), lambda b, i: (b, i, 0)),
        out_shape=jax.ShapeDtypeStruct((B, S, C), x.dtype),
    )(pattern, x)


# speedup vs baseline: 1.3231x; 1.0088x over previous
import functools

import numpy as np
import jax
import jax.numpy as jnp
from jax.experimental import pallas as pl

_ROTATIONS = ((13, 15, 26, 6), (17, 29, 16, 24))
_PARITY = np.uint32(0x1BD11BDA)
_THRESH = 1677721


def _np_threefry2x32(k0, k1, x0, x1):
    k0 = np.uint32(k0)
    k1 = np.uint32(k1)
    ks = (k0, k1, np.uint32(k0 ^ k1 ^ _PARITY))
    x0 = (x0 + ks[0]).astype(np.uint32)
    x1 = (x1 + ks[1]).astype(np.uint32)
    for i in range(5):
        for r in _ROTATIONS[i % 2]:
            x0 = (x0 + x1).astype(np.uint32)
            x1 = ((x1 << np.uint32(r)) | (x1 >> np.uint32(32 - r))).astype(np.uint32)
            x1 = x1 ^ x0
        x0 = (x0 + ks[(i + 1) % 3]).astype(np.uint32)
        x1 = (x1 + ks[(i + 2) % 3] + np.uint32(i + 1)).astype(np.uint32)
    return x0, x1


def _child_keys(seed, num):
    lo = np.arange(num, dtype=np.uint32)
    hi = np.zeros(num, dtype=np.uint32)
    y0, y1 = _np_threefry2x32(np.uint32(seed >> 32), np.uint32(seed & 0xFFFFFFFF), hi, lo)
    return np.stack([y0, y1], axis=-1)


_KEYS = _child_keys(42, 4)


def _mask_mul_kernel(pat_ref, x_ref, o_ref, *, bs, C):
    b = pl.program_id(0)
    i = pl.program_id(1)

    k0 = jnp.uint32(_KEYS[0, 0])
    k1 = jnp.uint32(_KEYS[0, 1])
    for bb in range(1, _KEYS.shape[0]):
        k0 = jnp.where(b == bb, jnp.uint32(_KEYS[bb, 0]), k0)
        k1 = jnp.where(b == bb, jnp.uint32(_KEYS[bb, 1]), k1)
    ks2 = k0 ^ k1 ^ _PARITY
    ks = (k0, k1, ks2)

    base_plus_k1 = ks[1] + jnp.uint32(i * (bs * C))

    x1 = pat_ref[0] + base_plus_k1
    x0 = jnp.full((bs, C), jnp.uint32(0), dtype=jnp.uint32) + ks[0]
    for r in range(5):
        for rot in _ROTATIONS[r % 2]:
            x0 = x0 + x1
            x1 = (x1 << jnp.uint32(rot)) ^ (x1 >> jnp.uint32(32 - rot)) ^ x0
        x0 = x0 + ks[(r + 1) % 3]
        x1 = x1 + (ks[(r + 2) % 3] + jnp.uint32(r + 1))

    bits = x0 ^ x1
    keep = bits > jnp.uint32((_THRESH + 1) * 512 - 1)
    o_ref[0] = jnp.where(keep, x_ref[0], jnp.float32(0.0))


@jax.jit
def kernel(x):
    B, S, C = x.shape
    bs = 512
    grid = (B, S // bs)
    pattern = (
        jax.lax.broadcasted_iota(jnp.uint32, (1, bs, C), 1) * jnp.uint32(C)
        + jax.lax.broadcasted_iota(jnp.uint32, (1, bs, C), 2)
    )
    return pl.pallas_call(
        functools.partial(_mask_mul_kernel, bs=bs, C=C),
        grid=grid,
        in_specs=[
            pl.BlockSpec((1, bs, C), lambda b, i: (0, 0, 0)),
            pl.BlockSpec((1, bs, C), lambda b, i: (b, i, 0)),
        ],
        out_specs=pl.BlockSpec((1, bs, C), lambda b, i: (b, i, 0)),
        out_shape=jax.ShapeDtypeStruct((B, S, C), x.dtype),
    )(pattern, x)


# speedup vs baseline: 1.3234x; 1.0002x over previous
import functools

import numpy as np
import jax
import jax.numpy as jnp
from jax.experimental import pallas as pl

_ROTATIONS = ((13, 15, 26, 6), (17, 29, 16, 24))
_PARITY = np.uint32(0x1BD11BDA)
_THRESH = 1677721


def _np_threefry2x32(k0, k1, x0, x1):
    k0 = np.uint32(k0)
    k1 = np.uint32(k1)
    ks = (k0, k1, np.uint32(k0 ^ k1 ^ _PARITY))
    x0 = (x0 + ks[0]).astype(np.uint32)
    x1 = (x1 + ks[1]).astype(np.uint32)
    for i in range(5):
        for r in _ROTATIONS[i % 2]:
            x0 = (x0 + x1).astype(np.uint32)
            x1 = ((x1 << np.uint32(r)) | (x1 >> np.uint32(32 - r))).astype(np.uint32)
            x1 = x1 ^ x0
        x0 = (x0 + ks[(i + 1) % 3]).astype(np.uint32)
        x1 = (x1 + ks[(i + 2) % 3] + np.uint32(i + 1)).astype(np.uint32)
    return x0, x1


def _child_keys(seed, num):
    lo = np.arange(num, dtype=np.uint32)
    hi = np.zeros(num, dtype=np.uint32)
    y0, y1 = _np_threefry2x32(np.uint32(seed >> 32), np.uint32(seed & 0xFFFFFFFF), hi, lo)
    return np.stack([y0, y1], axis=-1)


_KEYS = _child_keys(42, 4)


def _mask_mul_kernel(pat_ref, x_ref, o_ref, *, bs, C):
    b = pl.program_id(0)
    i = pl.program_id(1)

    k0 = jnp.uint32(_KEYS[0, 0])
    k1 = jnp.uint32(_KEYS[0, 1])
    for bb in range(1, _KEYS.shape[0]):
        k0 = jnp.where(b == bb, jnp.uint32(_KEYS[bb, 0]), k0)
        k1 = jnp.where(b == bb, jnp.uint32(_KEYS[bb, 1]), k1)
    ks2 = k0 ^ k1 ^ _PARITY
    ks = (k0, k1, ks2)

    base_plus_k1 = ks[1] + jnp.uint32(i * (bs * C))

    x1 = pat_ref[0] + base_plus_k1
    x0 = jnp.full((bs, C), jnp.uint32(0), dtype=jnp.uint32) + ks[0]
    for r in range(5):
        for rot in _ROTATIONS[r % 2]:
            x0 = x0 + x1
            x1 = (x1 * jnp.uint32(1 << rot)) ^ (x1 >> jnp.uint32(32 - rot)) ^ x0
        x0 = x0 + ks[(r + 1) % 3]
        x1 = x1 + (ks[(r + 2) % 3] + jnp.uint32(r + 1))

    bits = x0 ^ x1
    keep = bits > jnp.uint32((_THRESH + 1) * 512 - 1)
    o_ref[0] = jnp.where(keep, x_ref[0], jnp.float32(0.0))


@jax.jit
def kernel(x):
    B, S, C = x.shape
    bs = 512
    grid = (B, S // bs)
    pattern = (
        jax.lax.broadcasted_iota(jnp.uint32, (1, bs, C), 1) * jnp.uint32(C)
        + jax.lax.broadcasted_iota(jnp.uint32, (1, bs, C), 2)
    )
    return pl.pallas_call(
        functools.partial(_mask_mul_kernel, bs=bs, C=C),
        grid=grid,
        in_specs=[
            pl.BlockSpec((1, bs, C), lambda b, i: (0, 0, 0)),
            pl.BlockSpec((1, bs, C), lambda b, i: (b, i, 0)),
        ],
        out_specs=pl.BlockSpec((1, bs, C), lambda b, i: (b, i, 0)),
        out_shape=jax.ShapeDtypeStruct((B, S, C), x.dtype),
    )(pattern, x)
